# baseline (speedup 1.0000x reference)
import jax
import jax.numpy as jnp
from jax import lax
from jax.experimental import pallas as pl
from jax.experimental.pallas import tpu as pltpu
from jax.experimental.pallas import tpu_sc as plsc

VOCAB = 1000000
EMBED_DIM = 64
MAX_SEQ_LEN = 200
SEQ_LEN = 200
BATCH = 4096

NC = 2
NS = 16
NW = NC * NS
L = 16
NV = EMBED_DIM // L
NT = EMBED_DIM // 8

SEQ_PER_W = BATCH // NW
SB = 2
CHUNK = SB * SEQ_PER_W
N_CHUNKS = SEQ_LEN // SB

_EPS = 1e-5
_INV_D = 1.0 / EMBED_DIM

_GDN = lax.GatherDimensionNumbers(
    offset_dims=(), collapsed_slice_dims=(0,), start_index_map=(0,))


def _permute(v, perm):
    return lax.gather(v, perm[:, None], _GDN, (1,),
                      mode=lax.GatherScatterMode.PROMISE_IN_BOUNDS)


def _allsum(v, perms):
    for p in perms:
        v = v + _permute(v, p)
    return v


def _rsqrt(a):
    i = plsc.bitcast(a, jnp.int32)
    i = 0x5F3759DF - lax.shift_right_logical(i, 1)
    y = plsc.bitcast(i, jnp.float32)
    half_a = 0.5 * a
    for _ in range(2):
        y = y * (1.5 - half_a * y * y)
    return y


def _body(x_hbm, table_hbm, pos_hbm, gamma_hbm, beta_hbm, out_hbm,
          idx_v, idx_t, ib0, ib1, ob0, ob1, pb0, pb1, gb_v,
          gsem0, gsem1, wsem0, wsem1):
    wid = lax.axis_index("s") * NC + lax.axis_index("c")
    base_seq = wid * SEQ_PER_W

    pltpu.sync_copy(x_hbm.at[pl.ds(base_seq, SEQ_PER_W)], idx_v)
    pltpu.sync_copy(gamma_hbm, gb_v.at[0])
    pltpu.sync_copy(beta_hbm, gb_v.at[1])

    lanes = lax.iota(jnp.int32, L)
    perms = [lax.bitwise_xor(lanes, jnp.int32(1 << i)) for i in range(4)]
    g = [gb_v[0, pl.ds(i * L, L)] for i in range(NV)]
    b = [gb_v[1, pl.ds(i * L, L)] for i in range(NV)]
    sub_v = lax.bitwise_and(lanes, jnp.int32(7))
    td_v = [lax.shift_right_logical(lanes, 3) + jnp.int32(2 * i)
            for i in range(NV)]

    def tr_body(s, carry):
        for gi in range(SEQ_PER_W // L):
            col = lax.broadcast(s, (L,))
            v = plsc.load_gather(idx_v, [lanes + jnp.int32(gi * L), col])
            idx_t[s, pl.ds(gi * L, L)] = v
        return carry

    lax.fori_loop(0, SEQ_LEN, tr_body, 0)

    def fire_gather(k, ib, pb, gsem):
        s0 = k * SB
        for si in range(SB):
            pltpu.async_copy(table_hbm.at[idx_t.at[s0 + si]],
                             ib.at[pl.ds(si * SEQ_PER_W, SEQ_PER_W)], gsem)
        pltpu.async_copy(pos_hbm.at[0, pl.ds(s0, SB)], pb, gsem)

    def wait_gather(ib, pb, gsem):
        pltpu.make_async_copy(table_hbm.at[pl.ds(0, CHUNK)], ib, gsem).wait()
        pltpu.make_async_copy(pos_hbm.at[0, pl.ds(0, SB)], pb, gsem).wait()

    def fire_write(k, ob, wsem):
        pltpu.async_copy(ob, out_hbm.at[pl.ds(k * SB, SB), :, wid], wsem)

    def wait_write(ob, wsem):
        pltpu.make_async_copy(ob, out_hbm.at[pl.ds(0, SB), :, wid],
                              wsem).wait()

    def compute(ib, ob, pb):
        def row(r):
            sl = lax.shift_right_logical(r, 7)
            bl = lax.bitwise_and(r, jnp.int32(SEQ_PER_W - 1))
            sl_v = lax.broadcast(sl, (L,))
            bl_v = lax.broadcast(bl, (L,))
            h = [ib[r, pl.ds(i * L, L)] + pb[sl, pl.ds(i * L, L)]
                 for i in range(NV)]
            tot = (h[0] + h[1]) + (h[2] + h[3])
            sq = (h[0] * h[0] + h[1] * h[1]) + (h[2] * h[2] + h[3] * h[3])
            mean = _allsum(tot, perms) * _INV_D
            var = _allsum(sq, perms) * _INV_D - mean * mean
            rstd = _rsqrt(var + _EPS)
            off = mean * rstd
            for i in range(NV):
                res = (h[i] * rstd - off) * g[i] + b[i]
                plsc.store_scatter(ob, [sl_v, td_v[i], sub_v, bl_v], res)

        plsc.parallel_loop(0, CHUNK, unroll=4)(row)

    fire_gather(0, ib0, pb0, gsem0)
    fire_gather(1, ib1, pb1, gsem1)

    wait_gather(ib0, pb0, gsem0)
    compute(ib0, ob0, pb0)
    fire_write(0, ob0, wsem0)
    fire_gather(2, ib0, pb0, gsem0)

    wait_gather(ib1, pb1, gsem1)
    compute(ib1, ob1, pb1)
    fire_write(1, ob1, wsem1)
    fire_gather(3, ib1, pb1, gsem1)

    def pair_body(j, carry):
        k0 = 2 * j + 2
        wait_gather(ib0, pb0, gsem0)
        wait_write(ob0, wsem0)
        compute(ib0, ob0, pb0)
        fire_write(k0, ob0, wsem0)
        fire_gather(k0 + 2, ib0, pb0, gsem0)
        wait_gather(ib1, pb1, gsem1)
        wait_write(ob1, wsem1)
        compute(ib1, ob1, pb1)
        fire_write(k0 + 1, ob1, wsem1)
        fire_gather(k0 + 3, ib1, pb1, gsem1)
        return carry

    lax.fori_loop(0, (N_CHUNKS - 4) // 2, pair_body, 0)

    wait_gather(ib0, pb0, gsem0)
    wait_write(ob0, wsem0)
    compute(ib0, ob0, pb0)
    fire_write(N_CHUNKS - 2, ob0, wsem0)
    wait_gather(ib1, pb1, gsem1)
    wait_write(ob1, wsem1)
    compute(ib1, ob1, pb1)
    fire_write(N_CHUNKS - 1, ob1, wsem1)

    wait_write(ob0, wsem0)
    wait_write(ob1, wsem1)


@jax.jit
def _run(x, table, pos, gamma, beta):
    mesh = plsc.VectorSubcoreMesh(core_axis_name="c", subcore_axis_name="s",
                                  num_cores=NC, num_subcores=NS)
    f = pl.kernel(
        _body,
        out_type=jax.ShapeDtypeStruct((SEQ_LEN, NT, NW, 8, SEQ_PER_W),
                                      jnp.float32),
        mesh=mesh,
        compiler_params=pltpu.CompilerParams(needs_layout_passes=False,
                                             use_tc_tiling_on_sc=False),
        scratch_types=[
            pltpu.VMEM((SEQ_PER_W, SEQ_LEN), jnp.int32),
            pltpu.VMEM((SEQ_LEN, SEQ_PER_W), jnp.int32),
            pltpu.VMEM((CHUNK, EMBED_DIM), jnp.float32),
            pltpu.VMEM((CHUNK, EMBED_DIM), jnp.float32),
            pltpu.VMEM((SB, NT, 8, SEQ_PER_W), jnp.float32),
            pltpu.VMEM((SB, NT, 8, SEQ_PER_W), jnp.float32),
            pltpu.VMEM((SB, EMBED_DIM), jnp.float32),
            pltpu.VMEM((SB, EMBED_DIM), jnp.float32),
            pltpu.VMEM((2, EMBED_DIM), jnp.float32),
            pltpu.SemaphoreType.DMA,
            pltpu.SemaphoreType.DMA,
            pltpu.SemaphoreType.DMA,
            pltpu.SemaphoreType.DMA,
        ],
    )
    return f(x, table, pos, gamma, beta)


def kernel(x, token_table, pos_embed, gamma, beta):
    out5 = _run(x, token_table, pos_embed, gamma, beta)
    return out5.transpose(2, 4, 0, 1, 3).reshape(BATCH, SEQ_LEN, EMBED_DIM)


# speedup vs baseline: 1.6351x; 1.6351x over previous
import jax
import jax.numpy as jnp
from jax import lax
from jax.experimental import pallas as pl
from jax.experimental.pallas import tpu as pltpu
from jax.experimental.pallas import tpu_sc as plsc

VOCAB = 1000000
EMBED_DIM = 64
MAX_SEQ_LEN = 200
SEQ_LEN = 200
BATCH = 4096

NC = 2
NS = 16
NW = NC * NS
L = 16
NV = EMBED_DIM // L
NT = EMBED_DIM // 8

SEQ_PER_W = BATCH // NW
SB = 2
OB_PITCH = 133
CHUNK = SB * SEQ_PER_W
N_CHUNKS = SEQ_LEN // SB

_EPS = 1e-5
_INV_D = 1.0 / EMBED_DIM

_GDN = lax.GatherDimensionNumbers(
    offset_dims=(), collapsed_slice_dims=(0,), start_index_map=(0,))


def _permute(v, perm):
    return lax.gather(v, perm[:, None], _GDN, (1,),
                      mode=lax.GatherScatterMode.PROMISE_IN_BOUNDS)


def _allsum(v, perms):
    for p in perms:
        v = v + _permute(v, p)
    return v


def _rsqrt(a):
    i = plsc.bitcast(a, jnp.int32)
    i = 0x5F3759DF - lax.shift_right_logical(i, 1)
    y = plsc.bitcast(i, jnp.float32)
    half_a = 0.5 * a
    for _ in range(2):
        y = y * (1.5 - half_a * y * y)
    return y


def _body(x_hbm, table_hbm, pos_hbm, gamma_hbm, beta_hbm, out_hbm,
          idx_v, idx_t, ib0, ib1, ob0, ob1, pb0, pb1, gb_v,
          gsem0, gsem1, wsem0, wsem1):
    wid = lax.axis_index("s") * NC + lax.axis_index("c")
    base_seq = wid * SEQ_PER_W

    pltpu.sync_copy(x_hbm.at[pl.ds(base_seq, SEQ_PER_W)], idx_v)
    pltpu.sync_copy(gamma_hbm, gb_v.at[0])
    pltpu.sync_copy(beta_hbm, gb_v.at[1])

    lanes = lax.iota(jnp.int32, L)
    perms = [lax.bitwise_xor(lanes, jnp.int32(1 << i)) for i in range(4)]
    g = [gb_v[0, pl.ds(i * L, L)] for i in range(NV)]
    b = [gb_v[1, pl.ds(i * L, L)] for i in range(NV)]
    sub_v = lax.bitwise_and(lanes, jnp.int32(7))
    td_v = [lax.shift_right_logical(lanes, 3) + jnp.int32(2 * i)
            for i in range(NV)]

    def tr_body(s, carry):
        for gi in range(SEQ_PER_W // L):
            col = lax.broadcast(s, (L,))
            v = plsc.load_gather(idx_v, [lanes + jnp.int32(gi * L), col])
            idx_t[s, pl.ds(gi * L, L)] = v
        return carry

    lax.fori_loop(0, SEQ_LEN, tr_body, 0)

    def fire_gather(k, ib, pb, gsem):
        s0 = k * SB
        for si in range(SB):
            pltpu.async_copy(table_hbm.at[idx_t.at[s0 + si]],
                             ib.at[pl.ds(si * SEQ_PER_W, SEQ_PER_W)], gsem)
        pltpu.async_copy(pos_hbm.at[0, pl.ds(s0, SB)], pb, gsem)

    def wait_gather(ib, pb, gsem):
        pltpu.make_async_copy(table_hbm.at[pl.ds(0, CHUNK)], ib, gsem).wait()
        pltpu.make_async_copy(pos_hbm.at[0, pl.ds(0, SB)], pb, gsem).wait()

    def fire_write(k, ob, wsem):
        pltpu.async_copy(ob.at[:, :, :, pl.ds(0, SEQ_PER_W)],
                         out_hbm.at[pl.ds(k * SB, SB), :, wid], wsem)

    def wait_write(ob, wsem):
        pltpu.make_async_copy(ob.at[:, :, :, pl.ds(0, SEQ_PER_W)],
                              out_hbm.at[pl.ds(0, SB), :, wid],
                              wsem).wait()

    def compute(ib, ob, pb):
        def row(r):
            sl = lax.shift_right_logical(r, 7)
            bl = lax.bitwise_and(r, jnp.int32(SEQ_PER_W - 1))
            sl_v = lax.broadcast(sl, (L,))
            bl_v = lax.broadcast(bl, (L,))
            h = [ib[r, pl.ds(i * L, L)] + pb[sl, pl.ds(i * L, L)]
                 for i in range(NV)]
            tot = (h[0] + h[1]) + (h[2] + h[3])
            sq = (h[0] * h[0] + h[1] * h[1]) + (h[2] * h[2] + h[3] * h[3])
            mean = _allsum(tot, perms) * _INV_D
            var = _allsum(sq, perms) * _INV_D - mean * mean
            rstd = _rsqrt(var + _EPS)
            off = mean * rstd
            for i in range(NV):
                res = (h[i] * rstd - off) * g[i] + b[i]
                plsc.store_scatter(ob, [sl_v, td_v[i], sub_v, bl_v], res)

        plsc.parallel_loop(0, CHUNK, unroll=4)(row)

    fire_gather(0, ib0, pb0, gsem0)
    fire_gather(1, ib1, pb1, gsem1)

    wait_gather(ib0, pb0, gsem0)
    compute(ib0, ob0, pb0)
    fire_write(0, ob0, wsem0)
    fire_gather(2, ib0, pb0, gsem0)

    wait_gather(ib1, pb1, gsem1)
    compute(ib1, ob1, pb1)
    fire_write(1, ob1, wsem1)
    fire_gather(3, ib1, pb1, gsem1)

    def pair_body(j, carry):
        k0 = 2 * j + 2
        wait_gather(ib0, pb0, gsem0)
        wait_write(ob0, wsem0)
        compute(ib0, ob0, pb0)
        fire_write(k0, ob0, wsem0)
        fire_gather(k0 + 2, ib0, pb0, gsem0)
        wait_gather(ib1, pb1, gsem1)
        wait_write(ob1, wsem1)
        compute(ib1, ob1, pb1)
        fire_write(k0 + 1, ob1, wsem1)
        fire_gather(k0 + 3, ib1, pb1, gsem1)
        return carry

    lax.fori_loop(0, (N_CHUNKS - 4) // 2, pair_body, 0)

    wait_gather(ib0, pb0, gsem0)
    wait_write(ob0, wsem0)
    compute(ib0, ob0, pb0)
    fire_write(N_CHUNKS - 2, ob0, wsem0)
    wait_gather(ib1, pb1, gsem1)
    wait_write(ob1, wsem1)
    compute(ib1, ob1, pb1)
    fire_write(N_CHUNKS - 1, ob1, wsem1)

    wait_write(ob0, wsem0)
    wait_write(ob1, wsem1)


@jax.jit
def _run(x, table, pos, gamma, beta):
    mesh = plsc.VectorSubcoreMesh(core_axis_name="c", subcore_axis_name="s",
                                  num_cores=NC, num_subcores=NS)
    f = pl.kernel(
        _body,
        out_type=jax.ShapeDtypeStruct((SEQ_LEN, NT, NW, 8, SEQ_PER_W),
                                      jnp.float32),
        mesh=mesh,
        compiler_params=pltpu.CompilerParams(needs_layout_passes=False,
                                             use_tc_tiling_on_sc=False),
        scratch_types=[
            pltpu.VMEM((SEQ_PER_W, SEQ_LEN), jnp.int32),
            pltpu.VMEM((SEQ_LEN, SEQ_PER_W), jnp.int32),
            pltpu.VMEM((CHUNK, EMBED_DIM), jnp.float32),
            pltpu.VMEM((CHUNK, EMBED_DIM), jnp.float32),
            pltpu.VMEM((SB, NT, 8, OB_PITCH), jnp.float32),
            pltpu.VMEM((SB, NT, 8, OB_PITCH), jnp.float32),
            pltpu.VMEM((SB, EMBED_DIM), jnp.float32),
            pltpu.VMEM((SB, EMBED_DIM), jnp.float32),
            pltpu.VMEM((2, EMBED_DIM), jnp.float32),
            pltpu.SemaphoreType.DMA,
            pltpu.SemaphoreType.DMA,
            pltpu.SemaphoreType.DMA,
            pltpu.SemaphoreType.DMA,
        ],
    )
    return f(x, table, pos, gamma, beta)


def kernel(x, token_table, pos_embed, gamma, beta):
    out5 = _run(x, token_table, pos_embed, gamma, beta)
    return out5.transpose(2, 4, 0, 1, 3).reshape(BATCH, SEQ_LEN, EMBED_DIM)
